# Initial kernel scaffold; baseline (speedup 1.0000x reference)
import jax
import jax.numpy as jnp
from jax.experimental import pallas as pl


def kernel(latents, code_embed):
    raise NotImplementedError("write your pallas kernel here")


# speedup vs baseline: 1.7467x; 1.7467x over previous
import jax
import jax.numpy as jnp
from jax.experimental import pallas as pl

B, D, T = 32, 128, 4096
K = 1024
TBLK = 512


def _knn_kernel(lat_ref, cb_ref, quant_ref, codes_ref):
    lat = lat_ref[0]
    cb = cb_ref[...]

    xwT = jax.lax.dot_general(
        cb, lat, (((1,), (0,)), ((), ())),
        preferred_element_type=jnp.float32)

    x2 = jnp.sum(lat * lat, axis=0, keepdims=True)
    w2 = jnp.sum(cb * cb, axis=1, keepdims=True)
    d2T = (x2 - 2.0 * xwT) + w2

    m = jnp.min(d2T, axis=0, keepdims=True)
    ids = jax.lax.broadcasted_iota(jnp.int32, d2T.shape, 0)
    cand = jnp.where(d2T == m, ids, K)
    code = jnp.min(cand, axis=0, keepdims=True)
    codes_ref[0, 0, :] = code[0]

    ohT = (ids == code).astype(jnp.float32)
    quant = jax.lax.dot_general(
        ohT, cb, (((0,), (0,)), ((), ())),
        preferred_element_type=jnp.float32)
    quant_ref[0] = quant


def kernel(latents, code_embed):
    grid = (B, T // TBLK)
    quant, codes3 = pl.pallas_call(
        _knn_kernel,
        grid=grid,
        in_specs=[
            pl.BlockSpec((1, D, TBLK), lambda b, t: (b, 0, t)),
            pl.BlockSpec((K, D), lambda b, t: (0, 0)),
        ],
        out_specs=[
            pl.BlockSpec((1, TBLK, D), lambda b, t: (b, t, 0)),
            pl.BlockSpec((1, 1, TBLK), lambda b, t: (b, 0, t)),
        ],
        out_shape=[
            jax.ShapeDtypeStruct((B, T, D), jnp.float32),
            jax.ShapeDtypeStruct((B, 1, T), jnp.int32),
        ],
    )(latents, code_embed)
    return quant, codes3.reshape(B, T)
